# Initial kernel scaffold; baseline (speedup 1.0000x reference)
import jax
import jax.numpy as jnp
from jax.experimental import pallas as pl


def kernel(image, psf_rows, psf_cols, psf_vals, mat_z):
    raise NotImplementedError("write your pallas kernel here")


# speedup vs baseline: 1.4951x; 1.4951x over previous
import functools

import jax
import jax.numpy as jnp
from jax import lax
from jax.experimental import pallas as pl
from jax.experimental.pallas import tpu as pltpu
from jax.experimental.pallas import tpu_sc as plsc

N = 128 * 128
NZ = 1024
NCHUNK = 16
CZ = NZ // NCHUNK
NC, NS = 2, 16
B = 128
NNZ = 268435


def _matmul_body(x_ref, mz_ref, out_ref):
    r = lax.dot_general(
        x_ref[...], mz_ref[...], (((1,), (1,)), ((), ())),
        preferred_element_type=jnp.float32)
    for c in range(NCHUNK):
        out_ref[c] = r[:, c * CZ:(c + 1) * CZ]


def _matmul_chunked(squ, mat_z):
    BN = 2048
    return pl.pallas_call(
        _matmul_body,
        grid=(N // BN,),
        in_specs=[
            pl.BlockSpec((BN, NZ), lambda i: (i, 0)),
            pl.BlockSpec((NZ, NZ), lambda i: (0, 0)),
        ],
        out_specs=pl.BlockSpec((NCHUNK, BN, CZ), lambda i: (0, i, 0)),
        out_shape=jax.ShapeDtypeStruct((NCHUNK, N, CZ), jnp.float32),
    )(squ, mat_z)


def _sc_scatter(r1_flat, rows_p, cols_p, vals_p, nb):
    ew = nb * B
    zr = N // NS
    mesh = plsc.VectorSubcoreMesh(core_axis_name="c", subcore_axis_name="s")

    @functools.partial(
        pl.kernel,
        out_type=jax.ShapeDtypeStruct((NCHUNK, N, CZ), jnp.float32),
        mesh=mesh,
        scratch_types=[
            pltpu.VMEM((ew,), jnp.int32),
            pltpu.VMEM((ew,), jnp.int32),
            pltpu.VMEM((B,), jnp.int32),
            pltpu.VMEM((B,), jnp.int32),
            pltpu.VMEM((B, CZ), jnp.float32),
            pltpu.VMEM((zr // 8, CZ), jnp.float32),
            pltpu.VMEM((B,), jnp.float32),
            pltpu.VMEM_SHARED((N, CZ), jnp.float32),
            pltpu.SemaphoreType.DMA,
        ],
        compiler_params=pltpu.CompilerParams(use_tc_tiling_on_sc=False),
    )
    def k(r1_hbm, rows_hbm, cols_hbm, vals_hbm, out_hbm,
          rows_v, cols_v, idx_v, colsb_v, buf_v, z_v, vals_b,
          acc, sem):
        cid = lax.axis_index("c")
        sid = lax.axis_index("s")

        base = sid * ew
        pltpu.sync_copy(rows_hbm.at[pl.ds(base, ew)], rows_v)
        pltpu.sync_copy(cols_hbm.at[pl.ds(base, ew)], cols_v)

        zero16 = jnp.zeros((16,), jnp.float32)

        def zfill(i, _):
            for j in range(CZ // 16):
                z_v[i, pl.ds(j * 16, 16)] = zero16
            return 0

        lax.fori_loop(0, zr // 8, zfill, 0)

        def chunk_body(cc, _):
            c = cid * (NCHUNK // NC) + cc
            for h in range(8):
                pltpu.sync_copy(
                    z_v, acc.at[pl.ds(sid * zr + h * (zr // 8), zr // 8)])
            plsc.subcore_barrier()

            coff = c * N

            def blk_body(b, _):
                bb = b * B
                for g in range(B // 16):
                    sl = pl.ds(g * 16, 16)
                    idx_v[sl] = rows_v[pl.ds(bb + g * 16, 16)] + coff
                    colsb_v[sl] = cols_v[pl.ds(bb + g * 16, 16)]
                pltpu.sync_copy(vals_hbm.at[pl.ds(base + bb, B)], vals_b)
                pltpu.async_copy(r1_hbm.at[idx_v], buf_v, sem).wait()

                def ent_body(g, _):
                    vg = vals_b[pl.ds(g * 16, 16)]
                    for i16 in range(16):
                        vv = vg[i16]
                        i = g * 16 + i16
                        for j in range(CZ // 16):
                            sl = pl.ds(j * 16, 16)
                            buf_v[i, sl] = buf_v[i, sl] * vv
                    return 0

                lax.fori_loop(0, B // 16, ent_body, 0)
                pltpu.sync_copy(buf_v, acc.at[colsb_v], add=True)
                return 0

            lax.fori_loop(0, nb, blk_body, 0)
            plsc.subcore_barrier()
            pltpu.sync_copy(acc.at[pl.ds(sid * zr, zr)],
                            out_hbm.at[c, pl.ds(sid * zr, zr)])
            plsc.subcore_barrier()
            return 0

        lax.fori_loop(0, NCHUNK // NC, chunk_body, 0)

    return k(r1_flat, rows_p, cols_p, vals_p)


def kernel(image, psf_rows, psf_cols, psf_vals, mat_z):
    squ = image.reshape(N, NZ)
    r1c = _matmul_chunked(squ, mat_z)
    r1_flat = r1c.reshape(NCHUNK * N, CZ)

    nb = -(-NNZ // (NS * B))
    nnz_pad = NS * nb * B
    pad = nnz_pad - NNZ
    rows_p = jnp.pad(psf_rows, (0, pad))
    cols_p = jnp.pad(psf_cols, (0, pad))
    vals_p = jnp.pad(psf_vals, (0, pad))

    outc = _sc_scatter(r1_flat, rows_p, cols_p, vals_p, nb)
    return outc.transpose(1, 0, 2).reshape(128, 128, NZ)


# speedup vs baseline: 3.7940x; 2.5376x over previous
import functools

import jax
import jax.numpy as jnp
from jax import lax
from jax.experimental import pallas as pl
from jax.experimental.pallas import tpu as pltpu
from jax.experimental.pallas import tpu_sc as plsc

N = 128 * 128
NZ = 1024
NCHUNK = 16
CZ = NZ // NCHUNK
NC, NS = 2, 16
B = 128
NNZ = 268435


def _matmul_body(x_ref, mz_ref, out_ref):
    r = lax.dot_general(
        x_ref[...], mz_ref[...], (((1,), (1,)), ((), ())),
        preferred_element_type=jnp.float32)
    for c in range(NCHUNK):
        out_ref[c] = r[:, c * CZ:(c + 1) * CZ]


def _matmul_chunked(squ, mat_z):
    BN = 2048
    return pl.pallas_call(
        _matmul_body,
        grid=(N // BN,),
        in_specs=[
            pl.BlockSpec((BN, NZ), lambda i: (i, 0)),
            pl.BlockSpec((NZ, NZ), lambda i: (0, 0)),
        ],
        out_specs=pl.BlockSpec((NCHUNK, BN, CZ), lambda i: (0, i, 0)),
        out_shape=jax.ShapeDtypeStruct((NCHUNK, N, CZ), jnp.float32),
    )(squ, mat_z)


def _sc_scatter(r1_flat, rows_p, cols_p, vals_p, nb):
    ew = nb * B
    zr = N // NS
    mesh = plsc.VectorSubcoreMesh(core_axis_name="c", subcore_axis_name="s")

    zb = 64

    @functools.partial(
        pl.kernel,
        out_type=jax.ShapeDtypeStruct((NCHUNK, N, CZ), jnp.float32),
        mesh=mesh,
        scratch_types=[
            pltpu.VMEM((ew,), jnp.int32),
            pltpu.VMEM((ew,), jnp.float32),
            pltpu.VMEM((B,), jnp.int32),
            pltpu.VMEM((B,), jnp.int32),
            pltpu.VMEM((B,), jnp.int32),
            pltpu.VMEM((B,), jnp.int32),
            pltpu.VMEM((B, CZ), jnp.float32),
            pltpu.VMEM((B, CZ), jnp.float32),
            pltpu.VMEM((zb, CZ), jnp.float32),
            pltpu.SemaphoreType.DMA,
            pltpu.SemaphoreType.DMA,
            pltpu.SemaphoreType.DMA,
            pltpu.SemaphoreType.DMA,
            pltpu.VMEM_SHARED((N, CZ), jnp.float32),
        ],
        compiler_params=pltpu.CompilerParams(use_tc_tiling_on_sc=False),
    )
    def k(r1_hbm, rows_hbm, cols_hbm, vals_hbm, out_hbm,
          rows_v, vals_v, idx0, idx1, cb0, cb1, buf0, buf1, z_v,
          sg0, sg1, sc0, sc1, acc):
        cid = lax.axis_index("c")
        sid = lax.axis_index("s")
        idx = (idx0, idx1)
        cb = (cb0, cb1)
        buf = (buf0, buf1)
        sg = (sg0, sg1)
        sc = (sc0, sc1)

        base = sid * ew
        pltpu.sync_copy(rows_hbm.at[pl.ds(base, ew)], rows_v)
        pltpu.sync_copy(vals_hbm.at[pl.ds(base, ew)], vals_v)

        zero16 = jnp.zeros((16,), jnp.float32)

        def zfill(i, _):
            for j in range(CZ // 16):
                z_v[i, pl.ds(j * 16, 16)] = zero16
            return 0

        lax.fori_loop(0, zb, zfill, 0)

        def chunk_body(cc, _):
            c = cid * (NCHUNK // NC) + cc
            for h in range(zr // zb):
                pltpu.sync_copy(z_v, acc.at[pl.ds(sid * zr + h * zb, zb)])
            plsc.subcore_barrier()

            coff = c * N

            def fire(b, s):
                bb = b * B
                for g in range(B // 16):
                    sl = pl.ds(g * 16, 16)
                    idx[s][sl] = rows_v[pl.ds(bb + g * 16, 16)] + coff
                pltpu.async_copy(cols_hbm.at[pl.ds(base + bb, B)],
                                 cb[s], sc[s])
                pltpu.async_copy(r1_hbm.at[idx[s]], buf[s], sg[s])

            def wait(s):
                pltpu.make_async_copy(r1_hbm.at[idx[s]], buf[s], sg[s]).wait()
                pltpu.make_async_copy(cols_hbm.at[pl.ds(0, B)],
                                      cb[s], sc[s]).wait()

            def process(b, s):
                bb = b * B

                def ent_body(g, _):
                    vg = vals_v[pl.ds(bb + g * 16, 16)]
                    for i16 in range(16):
                        vv = vg[i16]
                        i = g * 16 + i16
                        for j in range(CZ // 16):
                            sl = pl.ds(j * 16, 16)
                            buf[s][i, sl] = buf[s][i, sl] * vv
                    return 0

                lax.fori_loop(0, B // 16, ent_body, 0)
                pltpu.sync_copy(buf[s], acc.at[cb[s]], add=True)

            fire(0, 0)

            def blk2(b2, _):
                b = b2 * 2
                fire(b + 1, 1)
                wait(0)
                process(b, 0)

                @pl.when(b2 + 1 < nb // 2)
                def _():
                    fire(b + 2, 0)

                wait(1)
                process(b + 1, 1)
                return 0

            lax.fori_loop(0, nb // 2, blk2, 0)
            plsc.subcore_barrier()
            pltpu.sync_copy(acc.at[pl.ds(sid * zr, zr)],
                            out_hbm.at[c, pl.ds(sid * zr, zr)])
            plsc.subcore_barrier()
            return 0

        lax.fori_loop(0, NCHUNK // NC, chunk_body, 0)

    return k(r1_flat, rows_p, cols_p, vals_p)


def kernel(image, psf_rows, psf_cols, psf_vals, mat_z):
    squ = image.reshape(N, NZ)
    r1c = _matmul_chunked(squ, mat_z)
    r1_flat = r1c.reshape(NCHUNK * N, CZ)

    nb = -(-NNZ // (NS * B))
    nnz_pad = NS * nb * B
    pad = nnz_pad - NNZ
    rows_p = jnp.pad(psf_rows, (0, pad))
    cols_p = jnp.pad(psf_cols, (0, pad))
    vals_p = jnp.pad(psf_vals, (0, pad))

    outc = _sc_scatter(r1_flat, rows_p, cols_p, vals_p, nb)
    return outc.transpose(1, 0, 2).reshape(128, 128, NZ)
